# baseline (device time: 87647 ns/iter reference)
import jax
import jax.numpy as jnp
from jax import lax
from jax.experimental import pallas as pl
from jax.experimental.pallas import tpu as pltpu

N_DEV = 4
N_STRIP = 4


def kernel(x, w_mat, scale_x, scale_w):
    m_total, k_shard = x.shape
    k2, n = w_mat.shape
    assert k2 == k_shard
    m_per = m_total // N_DEV
    n2 = n // 2
    ns = n2 // N_STRIP

    def body(x_ref, w_ref, sx_ref, sw_ref, out_ref,
             comm_l, comm_r, x_stage, w_stage, wb,
             send_l, recv_l, send_r, recv_r, stage_sems, w_sem):
        my = lax.axis_index("i")
        left = lax.rem(my + N_DEV - 1, N_DEV)
        right = lax.rem(my + 1, N_DEV)

        def stage(c, slot):
            return pltpu.make_async_copy(
                x_ref.at[pl.ds(c * m_per, m_per), :],
                x_stage.at[slot],
                stage_sems.at[slot],
            )

        c_m1 = lax.rem(my + N_DEV - 1, N_DEV)
        c_p1 = lax.rem(my + 1, N_DEV)
        c_p2 = lax.rem(my + 2, N_DEV)
        s0 = stage(c_m1, 0)
        s0.start()
        s1 = stage(c_p1, 1)
        s1.start()
        sw_l = pltpu.make_async_copy(w_ref.at[:, 0:n2], w_stage, w_sem)
        sw_r = pltpu.make_async_copy(w_ref.at[:, n2:], w_stage, w_sem)
        sw_l.start()

        barrier_sem = pltpu.get_barrier_semaphore()
        pl.semaphore_signal(barrier_sem, inc=1, device_id=(left,),
                            device_id_type=pl.DeviceIdType.MESH)
        pl.semaphore_signal(barrier_sem, inc=1, device_id=(right,),
                            device_id_type=pl.DeviceIdType.MESH)

        def cw_rdma(h, s):
            return pltpu.make_async_remote_copy(
                src_ref=comm_l.at[h % 2, :, s * ns:(s + 1) * ns],
                dst_ref=comm_l.at[(h + 1) % 2, :, s * ns:(s + 1) * ns],
                send_sem=send_l.at[h % 2, s],
                recv_sem=recv_l.at[(h + 1) % 2, s],
                device_id=(right,),
                device_id_type=pl.DeviceIdType.MESH,
            )

        def ccw_rdma(h, s):
            return pltpu.make_async_remote_copy(
                src_ref=comm_r.at[h % 2, :, s * ns:(s + 1) * ns],
                dst_ref=comm_r.at[(h + 1) % 2, :, s * ns:(s + 1) * ns],
                send_sem=send_r.at[h % 2, s],
                recv_sem=recv_r.at[(h + 1) % 2, s],
                device_id=(left,),
                device_id_type=pl.DeviceIdType.MESH,
            )

        sw_l.wait()
        wb[:, 0:n2] = w_stage[...].astype(jnp.bfloat16)
        sw_r.start()

        s0.wait()
        xb0 = x_stage[0].astype(jnp.bfloat16)
        comm_l[0, :, 0:ns] = jnp.dot(
            xb0, wb[:, 0:ns], preferred_element_type=jnp.float32
        ).astype(jnp.bfloat16)
        pl.semaphore_wait(barrier_sem, 2)
        cw_rdma(0, 0).start()
        sw_r.wait()
        wb[:, n2:] = w_stage[...].astype(jnp.bfloat16)
        s1.wait()
        xb1 = x_stage[1].astype(jnp.bfloat16)
        comm_r[0, :, 0:ns] = jnp.dot(
            xb1, wb[:, n2:n2 + ns], preferred_element_type=jnp.float32
        ).astype(jnp.bfloat16)
        ccw_rdma(0, 0).start()
        for s in range(1, N_STRIP):
            sl = slice(s * ns, (s + 1) * ns)
            comm_l[0, :, sl] = jnp.dot(
                xb0, wb[:, sl], preferred_element_type=jnp.float32
            ).astype(jnp.bfloat16)
            cw_rdma(0, s).start()
            comm_r[0, :, sl] = jnp.dot(
                xb1, wb[:, n2 + s * ns:n2 + (s + 1) * ns],
                preferred_element_type=jnp.float32
            ).astype(jnp.bfloat16)
            ccw_rdma(0, s).start()

        s2 = stage(c_p2, 0)
        s2.start()

        for h in range(N_DEV - 2):
            r_slot = (h + 1) % 2
            if h == 0:
                s2.wait()
                xc = x_stage[0].astype(jnp.bfloat16)
                pl_val = jnp.dot(xc, wb[:, :n2],
                                 preferred_element_type=jnp.float32)
            else:
                pl_val = jnp.dot(x_stage[1].astype(jnp.bfloat16),
                                 wb[:, :n2],
                                 preferred_element_type=jnp.float32)
            for s in range(N_STRIP):
                sl = slice(s * ns, (s + 1) * ns)
                cw_rdma(h, s).wait()
                comm_l[r_slot, :, sl] = (
                    comm_l[r_slot, :, sl].astype(jnp.float32)
                    + pl_val[:, sl]
                ).astype(jnp.bfloat16)
                cw_rdma(h + 1, s).start()
            if h == 0:
                pr_val = jnp.dot(xc, wb[:, n2:],
                                 preferred_element_type=jnp.float32)
                s3 = stage(c_m1, 0)
                s3.start()
            else:
                s3.wait()
                pr_val = jnp.dot(x_stage[0].astype(jnp.bfloat16),
                                 wb[:, n2:],
                                 preferred_element_type=jnp.float32)
                s4 = stage(my, 1)
                s4.start()
            for s in range(N_STRIP):
                sl = slice(s * ns, (s + 1) * ns)
                ccw_rdma(h, s).wait()
                comm_r[r_slot, :, sl] = (
                    comm_r[r_slot, :, sl].astype(jnp.float32)
                    + pr_val[:, sl]
                ).astype(jnp.bfloat16)
                ccw_rdma(h + 1, s).start()

        h = N_DEV - 2
        r_slot = (h + 1) % 2
        s4.wait()
        xc = x_stage[1].astype(jnp.bfloat16)
        pl_val = jnp.dot(xc, wb[:, :n2], preferred_element_type=jnp.float32)
        scale = sx_ref[0] * sw_ref[0]
        for s in range(N_STRIP):
            sl = slice(s * ns, (s + 1) * ns)
            cw_rdma(h, s).wait()
            x_stage[0, :, sl] = (
                comm_l[r_slot, :, sl].astype(jnp.float32) + pl_val[:, sl]
            ) * scale
        out_l = pltpu.make_async_copy(
            x_stage.at[0], out_ref.at[:, 0:n2], stage_sems.at[0])
        out_l.start()
        pr_val = jnp.dot(xc, wb[:, n2:], preferred_element_type=jnp.float32)
        for s in range(N_STRIP):
            sl = slice(s * ns, (s + 1) * ns)
            ccw_rdma(h, s).wait()
            x_stage[1, :, sl] = (
                comm_r[r_slot, :, sl].astype(jnp.float32) + pr_val[:, sl]
            ) * scale
        out_r = pltpu.make_async_copy(
            x_stage.at[1], out_ref.at[:, n2:], stage_sems.at[1])
        out_r.start()
        out_l.wait()
        out_r.wait()

    return pl.pallas_call(
        body,
        out_shape=jax.ShapeDtypeStruct((m_per, n), jnp.float32),
        in_specs=[
            pl.BlockSpec(memory_space=pl.ANY),
            pl.BlockSpec(memory_space=pl.ANY),
            pl.BlockSpec(memory_space=pltpu.SMEM),
            pl.BlockSpec(memory_space=pltpu.SMEM),
        ],
        out_specs=pl.BlockSpec(memory_space=pl.ANY),
        scratch_shapes=[
            pltpu.VMEM((2, m_per, n2), jnp.bfloat16),
            pltpu.VMEM((2, m_per, n2), jnp.bfloat16),
            pltpu.VMEM((2, m_per, k_shard), jnp.float32),
            pltpu.VMEM((k_shard, n2), jnp.float32),
            pltpu.VMEM((k_shard, n), jnp.bfloat16),
            pltpu.SemaphoreType.DMA((2, N_STRIP)),
            pltpu.SemaphoreType.DMA((2, N_STRIP)),
            pltpu.SemaphoreType.DMA((2, N_STRIP)),
            pltpu.SemaphoreType.DMA((2, N_STRIP)),
            pltpu.SemaphoreType.DMA((2,)),
            pltpu.SemaphoreType.DMA(()),
        ],
        compiler_params=pltpu.CompilerParams(collective_id=0),
    )(x, w_mat, scale_x, scale_w)


# device time: 83386 ns/iter; 1.0511x vs baseline; 1.0511x over previous
import jax
import jax.numpy as jnp
from jax import lax
from jax.experimental import pallas as pl
from jax.experimental.pallas import tpu as pltpu

N_DEV = 4
N_STRIP = 4


def kernel(x, w_mat, scale_x, scale_w):
    m_total, k_shard = x.shape
    k2, n = w_mat.shape
    assert k2 == k_shard
    m_per = m_total // N_DEV
    n2 = n // 2
    ns = n2 // N_STRIP

    wb = w_mat.astype(jnp.bfloat16)

    def body(x_ref, w_ref, sx_ref, sw_ref, out_ref,
             comm_l, comm_r, x_stage, send_l, recv_l, send_r, recv_r,
             stage_sems):
        my = lax.axis_index("i")
        left = lax.rem(my + N_DEV - 1, N_DEV)
        right = lax.rem(my + 1, N_DEV)

        def stage(c, slot):
            return pltpu.make_async_copy(
                x_ref.at[pl.ds(c * m_per, m_per), :],
                x_stage.at[slot],
                stage_sems.at[slot],
            )

        c_m1 = lax.rem(my + N_DEV - 1, N_DEV)
        c_p1 = lax.rem(my + 1, N_DEV)
        c_p2 = lax.rem(my + 2, N_DEV)
        s0 = stage(c_m1, 0)
        s0.start()
        s1 = stage(c_p1, 1)
        s1.start()

        barrier_sem = pltpu.get_barrier_semaphore()
        pl.semaphore_signal(barrier_sem, inc=1, device_id=(left,),
                            device_id_type=pl.DeviceIdType.MESH)
        pl.semaphore_signal(barrier_sem, inc=1, device_id=(right,),
                            device_id_type=pl.DeviceIdType.MESH)
        pl.semaphore_wait(barrier_sem, 2)

        def cw_rdma(h, s):
            return pltpu.make_async_remote_copy(
                src_ref=comm_l.at[h % 2, :, s * ns:(s + 1) * ns],
                dst_ref=comm_l.at[(h + 1) % 2, :, s * ns:(s + 1) * ns],
                send_sem=send_l.at[h % 2, s],
                recv_sem=recv_l.at[(h + 1) % 2, s],
                device_id=(right,),
                device_id_type=pl.DeviceIdType.MESH,
            )

        def ccw_rdma(h, s):
            return pltpu.make_async_remote_copy(
                src_ref=comm_r.at[h % 2, :, s * ns:(s + 1) * ns],
                dst_ref=comm_r.at[(h + 1) % 2, :, s * ns:(s + 1) * ns],
                send_sem=send_r.at[h % 2, s],
                recv_sem=recv_r.at[(h + 1) % 2, s],
                device_id=(left,),
                device_id_type=pl.DeviceIdType.MESH,
            )

        s0.wait()
        xb0 = x_stage[0].astype(jnp.bfloat16)
        comm_l[0, :, 0:ns] = jnp.dot(
            xb0, w_ref[:, 0:ns], preferred_element_type=jnp.float32
        ).astype(jnp.bfloat16)
        cw_rdma(0, 0).start()
        s1.wait()
        xb1 = x_stage[1].astype(jnp.bfloat16)
        comm_r[0, :, 0:ns] = jnp.dot(
            xb1, w_ref[:, n2:n2 + ns], preferred_element_type=jnp.float32
        ).astype(jnp.bfloat16)
        ccw_rdma(0, 0).start()
        for s in range(1, N_STRIP):
            sl = slice(s * ns, (s + 1) * ns)
            comm_l[0, :, sl] = jnp.dot(
                xb0, w_ref[:, sl], preferred_element_type=jnp.float32
            ).astype(jnp.bfloat16)
            cw_rdma(0, s).start()
            comm_r[0, :, sl] = jnp.dot(
                xb1, w_ref[:, n2 + s * ns:n2 + (s + 1) * ns],
                preferred_element_type=jnp.float32
            ).astype(jnp.bfloat16)
            ccw_rdma(0, s).start()

        s2 = stage(c_p2, 0)
        s2.start()

        for h in range(N_DEV - 1):
            r_slot = (h + 1) % 2
            if h == 0:
                s2.wait()
                xc = x_stage[0].astype(jnp.bfloat16)
                pl_val = jnp.dot(xc, w_ref[:, :n2],
                                 preferred_element_type=jnp.float32)
                pr_val = jnp.dot(xc, w_ref[:, n2:],
                                 preferred_element_type=jnp.float32)
                s3 = stage(c_m1, 0)
                s3.start()
            elif h == 1:
                pl_val = jnp.dot(x_stage[1].astype(jnp.bfloat16),
                                 w_ref[:, :n2],
                                 preferred_element_type=jnp.float32)
                s3.wait()
                pr_val = jnp.dot(x_stage[0].astype(jnp.bfloat16),
                                 w_ref[:, n2:],
                                 preferred_element_type=jnp.float32)
                s4 = stage(my, 1)
                s4.start()
            else:
                s4.wait()
                xc = x_stage[1].astype(jnp.bfloat16)
                pl_val = jnp.dot(xc, w_ref[:, :n2],
                                 preferred_element_type=jnp.float32)
                pr_val = jnp.dot(xc, w_ref[:, n2:],
                                 preferred_element_type=jnp.float32)
            last = h == N_DEV - 2
            if last:
                scale = sx_ref[0] * sw_ref[0]
            for s in range(N_STRIP):
                sl = slice(s * ns, (s + 1) * ns)
                cw_rdma(h, s).wait()
                if not last:
                    comm_l[r_slot, :, sl] = (
                        comm_l[r_slot, :, sl].astype(jnp.float32)
                        + pl_val[:, sl]
                    ).astype(jnp.bfloat16)
                    cw_rdma(h + 1, s).start()
                else:
                    out_ref[:, sl] = (
                        comm_l[r_slot, :, sl].astype(jnp.float32)
                        + pl_val[:, sl]
                    ) * scale
                ccw_rdma(h, s).wait()
                if not last:
                    comm_r[r_slot, :, sl] = (
                        comm_r[r_slot, :, sl].astype(jnp.float32)
                        + pr_val[:, sl]
                    ).astype(jnp.bfloat16)
                    ccw_rdma(h + 1, s).start()
                else:
                    out_ref[:, n2 + s * ns:n2 + (s + 1) * ns] = (
                        comm_r[r_slot, :, sl].astype(jnp.float32)
                        + pr_val[:, sl]
                    ) * scale

    return pl.pallas_call(
        body,
        out_shape=jax.ShapeDtypeStruct((m_per, n), jnp.float32),
        in_specs=[
            pl.BlockSpec(memory_space=pl.ANY),
            pl.BlockSpec(memory_space=pltpu.VMEM),
            pl.BlockSpec(memory_space=pltpu.SMEM),
            pl.BlockSpec(memory_space=pltpu.SMEM),
        ],
        out_specs=pl.BlockSpec(memory_space=pltpu.VMEM),
        scratch_shapes=[
            pltpu.VMEM((2, m_per, n2), jnp.bfloat16),
            pltpu.VMEM((2, m_per, n2), jnp.bfloat16),
            pltpu.VMEM((2, m_per, k_shard), jnp.float32),
            pltpu.SemaphoreType.DMA((2, N_STRIP)),
            pltpu.SemaphoreType.DMA((2, N_STRIP)),
            pltpu.SemaphoreType.DMA((2, N_STRIP)),
            pltpu.SemaphoreType.DMA((2, N_STRIP)),
            pltpu.SemaphoreType.DMA((2,)),
        ],
        compiler_params=pltpu.CompilerParams(collective_id=0),
    )(x, wb, scale_x, scale_w)
